# baseline (speedup 1.0000x reference)
import functools

import jax
import jax.numpy as jnp
from jax import lax
from jax.experimental import pallas as pl
from jax.experimental.pallas import tpu as pltpu
from jax.experimental.pallas import tpu_sc as plsc

N = 10000
E = 160000
EMB = 512
REC = 256
HID = 1024

NP = 10240
EPAD = 163840
NTILE = 16
EPT = EPAD // NTILE
CH = 128
NCH = EPT // CH
NST = 2
NCHS = NCH // NST
ROWS_PT = NP // NTILE
NBUF = 2
BLK = 512

_mesh = plsc.VectorSubcoreMesh(core_axis_name="c", subcore_axis_name="s")


@functools.partial(
    pl.kernel,
    mesh=_mesh,
    out_type=jax.ShapeDtypeStruct((NP,), jnp.float32),
    scratch_types=[
        pltpu.VMEM((EPT,), jnp.int32),
        pltpu.VMEM((NP,), jnp.float32),
        pltpu.VMEM((ROWS_PT,), jnp.float32),
        pltpu.VMEM((ROWS_PT,), jnp.float32),
        pltpu.VMEM_SHARED((NTILE, NP), jnp.float32),
    ],
    compiler_params=pltpu.CompilerParams(needs_layout_passes=False),
)
def _sc_deg(src_hbm, dis_hbm, src_v, deg_v, tot_v, tmp_v, red_sh):
    sid = lax.axis_index("s")
    pltpu.sync_copy(src_hbm.at[sid], src_v)

    zv = jnp.zeros((16,), jnp.float32)

    def _zero(i, _):
        deg_v[pl.ds(i * 16, 16)] = zv
        return ()

    lax.fori_loop(0, NP // 16, _zero, ())

    ones = jnp.ones((16,), jnp.float32)

    def _count(e, _):
        idx = src_v[pl.ds(e * 16, 16)]
        plsc.addupdate_scatter(deg_v, [idx], ones)
        return ()

    lax.fori_loop(0, EPT // 16, _count, ())

    pltpu.sync_copy(deg_v, red_sh.at[sid])
    plsc.subcore_barrier()

    base = sid * ROWS_PT
    pltpu.sync_copy(red_sh.at[0, pl.ds(base, ROWS_PT)], tot_v)

    def _accum_tile(t, _):
        pltpu.sync_copy(red_sh.at[t, pl.ds(base, ROWS_PT)], tmp_v)

        def _addv(k, _):
            sl = pl.ds(k * 16, 16)
            tot_v[sl] = tot_v[sl] + tmp_v[sl]
            return ()

        lax.fori_loop(0, ROWS_PT // 16, _addv, ())
        return ()

    lax.fori_loop(1, NTILE, _accum_tile, ())

    def _rsqrt(k, _):
        sl = pl.ds(k * 16, 16)
        d = tot_v[sl]
        i = plsc.bitcast(d, jnp.int32)
        i = 0x5F3759DF - lax.shift_right_logical(i, 1)
        y = plsc.bitcast(i, jnp.float32)
        y = y * (1.5 - 0.5 * d * y * y)
        y = y * (1.5 - 0.5 * d * y * y)
        y = y * (1.5 - 0.5 * d * y * y)
        tot_v[sl] = jnp.where(d > 0.5, y, 0.0)
        return ()

    lax.fori_loop(0, ROWS_PT // 16, _rsqrt, ())
    pltpu.sync_copy(tot_v, dis_hbm.at[pl.ds(base, ROWS_PT)])


@functools.partial(
    pl.kernel,
    mesh=_mesh,
    out_type=jax.ShapeDtypeStruct((2 * NP, 128), jnp.float32),
    scratch_types=[
        pltpu.VMEM((NCHS, CH), jnp.int32),
        pltpu.VMEM((NCHS, CH), jnp.int32),
        pltpu.VMEM((NBUF, CH, 128), jnp.float32),
        pltpu.VMEM_SHARED((NP, 128), jnp.float32),
        pltpu.SemaphoreType.DMA((NBUF,)),
        pltpu.SemaphoreType.DMA((NBUF,)),
    ],
    compiler_params=pltpu.CompilerParams(needs_layout_passes=False),
)
def _sc_agg(src_hbm, dst_hbm, gp_hbm, acc_hbm, src_v, dst_v, rows_v, acc_sh,
            gsems, ssems):
    cid = lax.axis_index("c")
    sid = lax.axis_index("s")

    zv = jnp.zeros((16,), jnp.float32)

    def _zero(i, _):
        rows_v[0, i // 8, pl.ds((i % 8) * 16, 16)] = zv
        return ()

    lax.fori_loop(0, CH * 8, _zero, ())
    for k in range(ROWS_PT // CH):
        pltpu.sync_copy(rows_v.at[0],
                        acc_sh.at[pl.ds(sid * ROWS_PT + k * CH, CH)])
    plsc.subcore_barrier()

    for h in range(NST):
        pltpu.sync_copy(src_hbm.at[cid, sid, pl.ds(h * NCHS, NCHS)], src_v)
        pltpu.sync_copy(dst_hbm.at[sid, pl.ds(h * NCHS, NCHS)], dst_v)
        pltpu.async_copy(gp_hbm.at[src_v.at[0]], rows_v.at[0], gsems.at[0])

        def _chunk(j, _):
            b = lax.rem(j, NBUF)
            b1 = 1 - b

            @pl.when(jnp.logical_and(j + 1 < NCHS, j >= 1))
            def _():
                pltpu.make_async_copy(rows_v.at[b1],
                                      acc_sh.at[dst_v.at[j - 1]],
                                      ssems.at[b1]).wait()

            @pl.when(j + 1 < NCHS)
            def _():
                pltpu.async_copy(gp_hbm.at[src_v.at[j + 1]], rows_v.at[b1],
                                 gsems.at[b1])

            pltpu.make_async_copy(gp_hbm.at[src_v.at[j]], rows_v.at[b],
                                  gsems.at[b]).wait()
            pltpu.async_copy(rows_v.at[b], acc_sh.at[dst_v.at[j]],
                             ssems.at[b], add=True)
            return ()

        lax.fori_loop(0, NCHS, _chunk, ())
        for j in (NCHS - 2, NCHS - 1):
            pltpu.make_async_copy(rows_v.at[j % NBUF],
                                  acc_sh.at[dst_v.at[j]],
                                  ssems.at[j % NBUF]).wait()
    plsc.subcore_barrier()

    pltpu.sync_copy(
        acc_sh.at[pl.ds(sid * ROWS_PT, ROWS_PT)],
        acc_hbm.at[pl.ds(cid * NP + sid * ROWS_PT, ROWS_PT)],
    )


def _dense_body(x_ref, w1_ref, b1_ref, w2_ref, b2_ref, w0_ref, wc1_ref,
                bc_ref, dis_ref, base_ref, gp_ref):
    h = lax.dot_general(x_ref[...], w1_ref[...], (((1,), (1,)), ((), ())),
                        preferred_element_type=jnp.float32)
    h = jnp.maximum(h + b1_ref[...], 0.0)
    h = lax.dot_general(h, w2_ref[...], (((1,), (1,)), ((), ())),
                        preferred_element_type=jnp.float32)
    h = jnp.maximum(h + b2_ref[...], 0.0)
    base_ref[...] = jnp.dot(h, w0_ref[...],
                            preferred_element_type=jnp.float32) + bc_ref[...]
    g = jnp.dot(h, wc1_ref[...], preferred_element_type=jnp.float32)
    gp = (-dis_ref[...]) * g
    gp_ref[0] = gp[:, :128]
    gp_ref[1] = gp[:, 128:]


def _dense(xp, w1, b1, w2, b2, w0, wc1, bc, dis1):
    nb = NP // BLK
    return pl.pallas_call(
        _dense_body,
        grid=(nb,),
        in_specs=[
            pl.BlockSpec((BLK, EMB), lambda i: (i, 0)),
            pl.BlockSpec((HID, EMB), lambda i: (0, 0)),
            pl.BlockSpec((1, HID), lambda i: (0, 0)),
            pl.BlockSpec((REC, HID), lambda i: (0, 0)),
            pl.BlockSpec((1, REC), lambda i: (0, 0)),
            pl.BlockSpec((REC, REC), lambda i: (0, 0)),
            pl.BlockSpec((REC, REC), lambda i: (0, 0)),
            pl.BlockSpec((1, REC), lambda i: (0, 0)),
            pl.BlockSpec((BLK, 1), lambda i: (i, 0)),
        ],
        out_specs=[
            pl.BlockSpec((BLK, REC), lambda i: (i, 0)),
            pl.BlockSpec((2, BLK, 128), lambda i: (0, i, 0)),
        ],
        out_shape=[
            jax.ShapeDtypeStruct((NP, REC), jnp.float32),
            jax.ShapeDtypeStruct((2, NP, 128), jnp.float32),
        ],
    )(xp, w1, b1, w2, b2, w0, wc1, bc, dis1)


def _combine_body(base_ref, acc_ref, dis_ref, out_ref):
    accf = jnp.concatenate([acc_ref[0], acc_ref[1]], axis=1)
    out_ref[...] = base_ref[...] + dis_ref[...] * accf


def _combine(base, acc3, dis1):
    nb = NP // BLK
    return pl.pallas_call(
        _combine_body,
        grid=(nb,),
        in_specs=[
            pl.BlockSpec((BLK, REC), lambda i: (i, 0)),
            pl.BlockSpec((2, BLK, 128), lambda i: (0, i, 0)),
            pl.BlockSpec((BLK, 1), lambda i: (i, 0)),
        ],
        out_specs=pl.BlockSpec((BLK, REC), lambda i: (i, 0)),
        out_shape=jax.ShapeDtypeStruct((NP, REC), jnp.float32),
    )(base, acc3, dis1)


def kernel(x, edge_index, W_fc1, b_fc1, W_fc2, b_fc2, W_cheb0, W_cheb1, b_cheb):
    xp = jnp.pad(x, ((0, NP - N), (0, 0)))
    src = edge_index[0]
    dst = edge_index[1]
    pad = jnp.full((EPAD - E,), NP - 1, jnp.int32)
    srcp = jnp.concatenate([src, pad])
    dstp = jnp.concatenate([dst, pad])

    src_a = srcp.reshape(NTILE, EPT)
    dis = _sc_deg(src_a)
    dis1 = dis.reshape(NP, 1)

    base, gp = _dense(xp, W_fc1, b_fc1.reshape(1, HID), W_fc2,
                      b_fc2.reshape(1, REC), W_cheb0, W_cheb1,
                      b_cheb.reshape(1, REC), dis1)

    src_c = jnp.stack([srcp, srcp + NP]).reshape(2, NTILE, NCH, CH)
    dst_c = dstp.reshape(NTILE, NCH, CH)
    acc = _sc_agg(src_c, dst_c, gp.reshape(2 * NP, 128))

    out = _combine(base, acc.reshape(2, NP, 128), dis1)
    return out[:N]


# speedup vs baseline: 1.3018x; 1.3018x over previous
import functools

import jax
import jax.numpy as jnp
from jax import lax
from jax.experimental import pallas as pl
from jax.experimental.pallas import tpu as pltpu
from jax.experimental.pallas import tpu_sc as plsc

N = 10000
E = 160000
EMB = 512
REC = 256
HID = 1024

NP = 10240
EPAD = 163840
NTILE = 16
EPT = EPAD // NTILE
CH = 128
NCH = EPT // CH
NST = 2
NCHS = NCH // NST
ROWS_PT = NP // NTILE
NBUF = 2
NPASS = 2
BLK = 512

_mesh = plsc.VectorSubcoreMesh(core_axis_name="c", subcore_axis_name="s")


@functools.partial(
    pl.kernel,
    mesh=_mesh,
    out_type=jax.ShapeDtypeStruct((NP,), jnp.float32),
    scratch_types=[
        pltpu.VMEM((EPT,), jnp.int32),
        pltpu.VMEM((NP,), jnp.float32),
        pltpu.VMEM((ROWS_PT,), jnp.float32),
        pltpu.VMEM((ROWS_PT,), jnp.float32),
        pltpu.VMEM_SHARED((NTILE, NP), jnp.float32),
    ],
    compiler_params=pltpu.CompilerParams(needs_layout_passes=False),
)
def _sc_deg(src_hbm, dis_hbm, src_v, deg_v, tot_v, tmp_v, red_sh):
    sid = lax.axis_index("s")
    pltpu.sync_copy(src_hbm.at[sid], src_v)

    zv = jnp.zeros((16,), jnp.float32)

    def _zero(i, _):
        deg_v[pl.ds(i * 16, 16)] = zv
        return ()

    lax.fori_loop(0, NP // 16, _zero, ())

    ones = jnp.ones((16,), jnp.float32)

    def _count(e, _):
        idx = src_v[pl.ds(e * 16, 16)]
        plsc.addupdate_scatter(deg_v, [idx], ones)
        return ()

    lax.fori_loop(0, EPT // 16, _count, ())

    pltpu.sync_copy(deg_v, red_sh.at[sid])
    plsc.subcore_barrier()

    base = sid * ROWS_PT
    pltpu.sync_copy(red_sh.at[0, pl.ds(base, ROWS_PT)], tot_v)

    def _accum_tile(t, _):
        pltpu.sync_copy(red_sh.at[t, pl.ds(base, ROWS_PT)], tmp_v)

        def _addv(k, _):
            sl = pl.ds(k * 16, 16)
            tot_v[sl] = tot_v[sl] + tmp_v[sl]
            return ()

        lax.fori_loop(0, ROWS_PT // 16, _addv, ())
        return ()

    lax.fori_loop(1, NTILE, _accum_tile, ())

    def _rsqrt(k, _):
        sl = pl.ds(k * 16, 16)
        d = tot_v[sl]
        i = plsc.bitcast(d, jnp.int32)
        i = 0x5F3759DF - lax.shift_right_logical(i, 1)
        y = plsc.bitcast(i, jnp.float32)
        y = y * (1.5 - 0.5 * d * y * y)
        y = y * (1.5 - 0.5 * d * y * y)
        y = y * (1.5 - 0.5 * d * y * y)
        tot_v[sl] = jnp.where(d > 0.5, y, 0.0)
        return ()

    lax.fori_loop(0, ROWS_PT // 16, _rsqrt, ())
    pltpu.sync_copy(tot_v, dis_hbm.at[pl.ds(base, ROWS_PT)])


QC = 128 // NPASS


@functools.partial(
    pl.kernel,
    mesh=_mesh,
    out_type=jax.ShapeDtypeStruct((2, NP, 128), jnp.float32),
    scratch_types=[
        pltpu.VMEM((NCHS, CH), jnp.int32),
        pltpu.VMEM((NCHS, CH), jnp.int32),
        pltpu.VMEM((NBUF, CH, QC), jnp.float32),
        pltpu.VMEM_SHARED((NP, QC), jnp.float32),
        pltpu.VMEM_SHARED((NP, QC), jnp.float32),
        pltpu.SemaphoreType.DMA((NBUF,)),
        pltpu.SemaphoreType.DMA((NBUF,)),
    ],
    compiler_params=pltpu.CompilerParams(needs_layout_passes=False,
                                         use_tc_tiling_on_sc=False),
)
def _sc_agg(src_hbm, dst_hbm, gp_hbm, acc_hbm, src_v, dst_v, rows_v, gp_sh,
            acc_sh, gsems, ssems):
    cid = lax.axis_index("c")
    sid = lax.axis_index("s")
    r0 = sid * ROWS_PT
    zv = jnp.zeros((16,), jnp.float32)

    for p in range(NPASS):
        pltpu.sync_copy(
            gp_hbm.at[cid, pl.ds(r0, ROWS_PT), pl.ds(QC * p, QC)],
            gp_sh.at[pl.ds(r0, ROWS_PT)])

        def _zero(i, _):
            rows_v[0, i // (QC // 16), pl.ds((i % (QC // 16)) * 16, 16)] = zv
            return ()

        lax.fori_loop(0, CH * (QC // 16), _zero, ())
        for k in range(ROWS_PT // CH):
            pltpu.sync_copy(rows_v.at[0],
                            acc_sh.at[pl.ds(r0 + k * CH, CH)])
        plsc.subcore_barrier()

        for h in range(NST):
            pltpu.sync_copy(src_hbm.at[sid, pl.ds(h * NCHS, NCHS)], src_v)
            pltpu.sync_copy(dst_hbm.at[sid, pl.ds(h * NCHS, NCHS)], dst_v)
            pltpu.async_copy(gp_sh.at[src_v.at[0]], rows_v.at[0], gsems.at[0])

            def _chunk(j, _):
                b = lax.rem(j, NBUF)
                b1 = 1 - b

                @pl.when(jnp.logical_and(j + 1 < NCHS, j >= 1))
                def _():
                    pltpu.make_async_copy(rows_v.at[b1],
                                          acc_sh.at[dst_v.at[j - 1]],
                                          ssems.at[b1]).wait()

                @pl.when(j + 1 < NCHS)
                def _():
                    pltpu.async_copy(gp_sh.at[src_v.at[j + 1]], rows_v.at[b1],
                                     gsems.at[b1])

                pltpu.make_async_copy(gp_sh.at[src_v.at[j]], rows_v.at[b],
                                      gsems.at[b]).wait()
                pltpu.async_copy(rows_v.at[b], acc_sh.at[dst_v.at[j]],
                                 ssems.at[b], add=True)
                return ()

            lax.fori_loop(0, NCHS, _chunk, ())
            for j in (NCHS - 2, NCHS - 1):
                pltpu.make_async_copy(rows_v.at[j % NBUF],
                                      acc_sh.at[dst_v.at[j]],
                                      ssems.at[j % NBUF]).wait()
        plsc.subcore_barrier()

        pltpu.sync_copy(
            acc_sh.at[pl.ds(r0, ROWS_PT)],
            acc_hbm.at[cid, pl.ds(r0, ROWS_PT), pl.ds(QC * p, QC)])
        plsc.subcore_barrier()


def _dense_body(x_ref, w1_ref, b1_ref, w2_ref, b2_ref, w0_ref, wc1_ref,
                bc_ref, dis_ref, base_ref, gp_ref):
    h = lax.dot_general(x_ref[...], w1_ref[...], (((1,), (1,)), ((), ())),
                        preferred_element_type=jnp.float32)
    h = jnp.maximum(h + b1_ref[...], 0.0)
    h = lax.dot_general(h, w2_ref[...], (((1,), (1,)), ((), ())),
                        preferred_element_type=jnp.float32)
    h = jnp.maximum(h + b2_ref[...], 0.0)
    base_ref[...] = jnp.dot(h, w0_ref[...],
                            preferred_element_type=jnp.float32) + bc_ref[...]
    g = jnp.dot(h, wc1_ref[...], preferred_element_type=jnp.float32)
    gp = (-dis_ref[...]) * g
    gp_ref[0] = gp[:, :128]
    gp_ref[1] = gp[:, 128:]


def _dense(xp, w1, b1, w2, b2, w0, wc1, bc, dis1):
    nb = NP // BLK
    return pl.pallas_call(
        _dense_body,
        grid=(nb,),
        in_specs=[
            pl.BlockSpec((BLK, EMB), lambda i: (i, 0)),
            pl.BlockSpec((HID, EMB), lambda i: (0, 0)),
            pl.BlockSpec((1, HID), lambda i: (0, 0)),
            pl.BlockSpec((REC, HID), lambda i: (0, 0)),
            pl.BlockSpec((1, REC), lambda i: (0, 0)),
            pl.BlockSpec((REC, REC), lambda i: (0, 0)),
            pl.BlockSpec((REC, REC), lambda i: (0, 0)),
            pl.BlockSpec((1, REC), lambda i: (0, 0)),
            pl.BlockSpec((BLK, 1), lambda i: (i, 0)),
        ],
        out_specs=[
            pl.BlockSpec((BLK, REC), lambda i: (i, 0)),
            pl.BlockSpec((2, BLK, 128), lambda i: (0, i, 0)),
        ],
        out_shape=[
            jax.ShapeDtypeStruct((NP, REC), jnp.float32),
            jax.ShapeDtypeStruct((2, NP, 128), jnp.float32),
        ],
    )(xp, w1, b1, w2, b2, w0, wc1, bc, dis1)


def _combine_body(base_ref, acc_ref, dis_ref, out_ref):
    accf = jnp.concatenate([acc_ref[0], acc_ref[1]], axis=1)
    out_ref[...] = base_ref[...] + dis_ref[...] * accf


def _combine(base, acc3, dis1):
    nb = NP // BLK
    return pl.pallas_call(
        _combine_body,
        grid=(nb,),
        in_specs=[
            pl.BlockSpec((BLK, REC), lambda i: (i, 0)),
            pl.BlockSpec((2, BLK, 128), lambda i: (0, i, 0)),
            pl.BlockSpec((BLK, 1), lambda i: (i, 0)),
        ],
        out_specs=pl.BlockSpec((BLK, REC), lambda i: (i, 0)),
        out_shape=jax.ShapeDtypeStruct((NP, REC), jnp.float32),
    )(base, acc3, dis1)


def kernel(x, edge_index, W_fc1, b_fc1, W_fc2, b_fc2, W_cheb0, W_cheb1, b_cheb):
    xp = jnp.pad(x, ((0, NP - N), (0, 0)))
    src = edge_index[0]
    dst = edge_index[1]
    pad = jnp.full((EPAD - E,), NP - 1, jnp.int32)
    srcp = jnp.concatenate([src, pad])
    dstp = jnp.concatenate([dst, pad])

    src_a = srcp.reshape(NTILE, EPT)
    dis = _sc_deg(src_a)
    dis1 = dis.reshape(NP, 1)

    base, gp = _dense(xp, W_fc1, b_fc1.reshape(1, HID), W_fc2,
                      b_fc2.reshape(1, REC), W_cheb0, W_cheb1,
                      b_cheb.reshape(1, REC), dis1)

    src_c = srcp.reshape(NTILE, NCH, CH)
    dst_c = dstp.reshape(NTILE, NCH, CH)
    acc = _sc_agg(src_c, dst_c, gp)

    out = _combine(base, acc, dis1)
    return out[:N]


# speedup vs baseline: 1.3713x; 1.0534x over previous
import functools

import jax
import jax.numpy as jnp
from jax import lax
from jax.experimental import pallas as pl
from jax.experimental.pallas import tpu as pltpu
from jax.experimental.pallas import tpu_sc as plsc

N = 10000
E = 160000
EMB = 512
REC = 256
HID = 1024

NP = 10240
EPAD = 163840
NTILE = 16
EPT = EPAD // NTILE
CH = 128
NCH = EPT // CH
NST = 1
NCHS = NCH // NST
ROWS_PT = NP // NTILE
NBUF = 3
NPASS = 2
BLK = 512
BLKC = 400

_mesh = plsc.VectorSubcoreMesh(core_axis_name="c", subcore_axis_name="s")


@functools.partial(
    pl.kernel,
    mesh=_mesh,
    out_type=jax.ShapeDtypeStruct((NP,), jnp.float32),
    scratch_types=[
        pltpu.VMEM((EPT,), jnp.int32),
        pltpu.VMEM((NP,), jnp.float32),
        pltpu.VMEM((ROWS_PT,), jnp.float32),
        pltpu.VMEM((ROWS_PT,), jnp.float32),
        pltpu.VMEM_SHARED((NTILE, NP), jnp.float32),
    ],
    compiler_params=pltpu.CompilerParams(needs_layout_passes=False),
)
def _sc_deg(src_hbm, dis_hbm, src_v, deg_v, tot_v, tmp_v, red_sh):
    sid = lax.axis_index("s")
    pltpu.sync_copy(src_hbm.at[sid], src_v)

    zv = jnp.zeros((16,), jnp.float32)

    def _zero(i, _):
        deg_v[pl.ds(i * 16, 16)] = zv
        return ()

    lax.fori_loop(0, NP // 16, _zero, ())

    ones = jnp.ones((16,), jnp.float32)

    def _count(e, _):
        idx = src_v[pl.ds(e * 16, 16)]
        plsc.addupdate_scatter(deg_v, [idx], ones)
        return ()

    lax.fori_loop(0, EPT // 16, _count, ())

    pltpu.sync_copy(deg_v, red_sh.at[sid])
    plsc.subcore_barrier()

    base = sid * ROWS_PT
    pltpu.sync_copy(red_sh.at[0, pl.ds(base, ROWS_PT)], tot_v)

    def _accum_tile(t, _):
        pltpu.sync_copy(red_sh.at[t, pl.ds(base, ROWS_PT)], tmp_v)

        def _addv(k, _):
            sl = pl.ds(k * 16, 16)
            tot_v[sl] = tot_v[sl] + tmp_v[sl]
            return ()

        lax.fori_loop(0, ROWS_PT // 16, _addv, ())
        return ()

    lax.fori_loop(1, NTILE, _accum_tile, ())

    def _rsqrt(k, _):
        sl = pl.ds(k * 16, 16)
        d = tot_v[sl]
        i = plsc.bitcast(d, jnp.int32)
        i = 0x5F3759DF - lax.shift_right_logical(i, 1)
        y = plsc.bitcast(i, jnp.float32)
        y = y * (1.5 - 0.5 * d * y * y)
        y = y * (1.5 - 0.5 * d * y * y)
        y = y * (1.5 - 0.5 * d * y * y)
        tot_v[sl] = jnp.where(d > 0.5, y, 0.0)
        return ()

    lax.fori_loop(0, ROWS_PT // 16, _rsqrt, ())
    pltpu.sync_copy(tot_v, dis_hbm.at[pl.ds(base, ROWS_PT)])


QC = 128 // NPASS


@functools.partial(
    pl.kernel,
    mesh=_mesh,
    out_type=jax.ShapeDtypeStruct((2, NP, 128), jnp.float32),
    scratch_types=[
        pltpu.VMEM((NCHS, CH), jnp.int32),
        pltpu.VMEM((NCHS, CH), jnp.int32),
        pltpu.VMEM((NBUF, CH, QC), jnp.float32),
        pltpu.VMEM_SHARED((NP, QC), jnp.float32),
        pltpu.VMEM_SHARED((NP, QC), jnp.float32),
        pltpu.SemaphoreType.DMA((NBUF,)),
        pltpu.SemaphoreType.DMA((NBUF,)),
    ],
    compiler_params=pltpu.CompilerParams(needs_layout_passes=False,
                                         use_tc_tiling_on_sc=False),
)
def _sc_agg(src_hbm, dst_hbm, gp_hbm, acc_hbm, src_v, dst_v, rows_v, gp_sh,
            acc_sh, gsems, ssems):
    cid = lax.axis_index("c")
    sid = lax.axis_index("s")
    r0 = sid * ROWS_PT
    zv = jnp.zeros((16,), jnp.float32)

    for p in range(NPASS):
        pltpu.sync_copy(
            gp_hbm.at[cid, pl.ds(r0, ROWS_PT), pl.ds(QC * p, QC)],
            gp_sh.at[pl.ds(r0, ROWS_PT)])

        def _zero(i, _):
            rows_v[0, i // (QC // 16), pl.ds((i % (QC // 16)) * 16, 16)] = zv
            return ()

        lax.fori_loop(0, CH * (QC // 16), _zero, ())
        for k in range(ROWS_PT // CH):
            pltpu.sync_copy(rows_v.at[0],
                            acc_sh.at[pl.ds(r0 + k * CH, CH)])
        plsc.subcore_barrier()

        if p == 0:
            pltpu.sync_copy(src_hbm.at[sid], src_v)
            pltpu.sync_copy(dst_hbm.at[sid], dst_v)
        for b in range(NBUF - 1):
            pltpu.async_copy(gp_sh.at[src_v.at[b]], rows_v.at[b], gsems.at[b])

        def _chunk(j, _):
            b = lax.rem(j, NBUF)
            jn = j + NBUF - 1
            bn = lax.rem(jn, NBUF)

            @pl.when(jnp.logical_and(jn < NCHS, j >= 1))
            def _():
                pltpu.make_async_copy(rows_v.at[bn],
                                      acc_sh.at[dst_v.at[j - 1]],
                                      ssems.at[bn]).wait()

            @pl.when(jn < NCHS)
            def _():
                pltpu.async_copy(gp_sh.at[src_v.at[jn]], rows_v.at[bn],
                                 gsems.at[bn])

            pltpu.make_async_copy(gp_sh.at[src_v.at[j]], rows_v.at[b],
                                  gsems.at[b]).wait()
            pltpu.async_copy(rows_v.at[b], acc_sh.at[dst_v.at[j]],
                             ssems.at[b], add=True)
            return ()

        lax.fori_loop(0, NCHS, _chunk, ())
        for j in range(NCHS - NBUF, NCHS):
            pltpu.make_async_copy(rows_v.at[j % NBUF],
                                  acc_sh.at[dst_v.at[j]],
                                  ssems.at[j % NBUF]).wait()
        plsc.subcore_barrier()

        pltpu.sync_copy(
            acc_sh.at[pl.ds(r0, ROWS_PT)],
            acc_hbm.at[cid, pl.ds(r0, ROWS_PT), pl.ds(QC * p, QC)])
        plsc.subcore_barrier()


def _dense_body(x_ref, w1_ref, b1_ref, w2_ref, b2_ref, w0_ref, wc1_ref,
                bc_ref, base_ref, g_ref):
    h = lax.dot_general(x_ref[...], w1_ref[...], (((1,), (1,)), ((), ())),
                        preferred_element_type=jnp.float32)
    h = jnp.maximum(h + b1_ref[...], 0.0)
    h = lax.dot_general(h, w2_ref[...], (((1,), (1,)), ((), ())),
                        preferred_element_type=jnp.float32)
    h = jnp.maximum(h + b2_ref[...], 0.0)
    base_ref[...] = jnp.dot(h, w0_ref[...],
                            preferred_element_type=jnp.float32) + bc_ref[...]
    g_ref[...] = jnp.dot(h, wc1_ref[...], preferred_element_type=jnp.float32)


def _dense(x, w1, b1, w2, b2, w0, wc1, bc):
    nb = NP // BLK
    return pl.pallas_call(
        _dense_body,
        grid=(nb,),
        in_specs=[
            pl.BlockSpec((BLK, EMB), lambda i: (i, 0)),
            pl.BlockSpec((HID, EMB), lambda i: (0, 0)),
            pl.BlockSpec((1, HID), lambda i: (0, 0)),
            pl.BlockSpec((REC, HID), lambda i: (0, 0)),
            pl.BlockSpec((1, REC), lambda i: (0, 0)),
            pl.BlockSpec((REC, REC), lambda i: (0, 0)),
            pl.BlockSpec((REC, REC), lambda i: (0, 0)),
            pl.BlockSpec((1, REC), lambda i: (0, 0)),
        ],
        out_specs=[
            pl.BlockSpec((BLK, REC), lambda i: (i, 0)),
            pl.BlockSpec((BLK, REC), lambda i: (i, 0)),
        ],
        out_shape=[
            jax.ShapeDtypeStruct((NP, REC), jnp.float32),
            jax.ShapeDtypeStruct((NP, REC), jnp.float32),
        ],
    )(x, w1, b1, w2, b2, w0, wc1, bc)


def _scale_body(g_ref, dis_ref, gp_ref):
    gp = (-dis_ref[...]) * g_ref[...]
    gp_ref[0] = gp[:, :128]
    gp_ref[1] = gp[:, 128:]


def _scale(g, dis1):
    nb = NP // BLK
    return pl.pallas_call(
        _scale_body,
        grid=(nb,),
        in_specs=[
            pl.BlockSpec((BLK, REC), lambda i: (i, 0)),
            pl.BlockSpec((BLK, 1), lambda i: (i, 0)),
        ],
        out_specs=pl.BlockSpec((2, BLK, 128), lambda i: (0, i, 0)),
        out_shape=jax.ShapeDtypeStruct((2, NP, 128), jnp.float32),
    )(g, dis1)


def _combine_body(base_ref, acc_ref, dis_ref, out_ref):
    accf = jnp.concatenate([acc_ref[0], acc_ref[1]], axis=1)
    out_ref[...] = base_ref[...] + dis_ref[...] * accf


def _combine(base, acc3, dis1):
    nb = N // BLKC
    return pl.pallas_call(
        _combine_body,
        grid=(nb,),
        in_specs=[
            pl.BlockSpec((BLKC, REC), lambda i: (i, 0)),
            pl.BlockSpec((2, BLKC, 128), lambda i: (0, i, 0)),
            pl.BlockSpec((BLKC, 1), lambda i: (i, 0)),
        ],
        out_specs=pl.BlockSpec((BLKC, REC), lambda i: (i, 0)),
        out_shape=jax.ShapeDtypeStruct((N, REC), jnp.float32),
    )(base, acc3, dis1)


def kernel(x, edge_index, W_fc1, b_fc1, W_fc2, b_fc2, W_cheb0, W_cheb1, b_cheb):
    src = edge_index[0]
    dst = edge_index[1]
    pad = jnp.full((EPAD - E,), NP - 1, jnp.int32)
    srcp = jnp.concatenate([src, pad])
    dstp = jnp.concatenate([dst, pad])

    src_a = srcp.reshape(NTILE, EPT)
    dis = _sc_deg(src_a)
    dis1 = dis.reshape(NP, 1)

    base, g = _dense(x, W_fc1, b_fc1.reshape(1, HID), W_fc2,
                     b_fc2.reshape(1, REC), W_cheb0, W_cheb1,
                     b_cheb.reshape(1, REC))

    gp = _scale(g, dis1)

    src_c = srcp.reshape(NTILE, NCH, CH)
    dst_c = dstp.reshape(NTILE, NCH, CH)
    acc = _sc_agg(src_c, dst_c, gp)

    return _combine(base, acc, dis1)
